# baseline (speedup 1.0000x reference)
import functools

import jax
import jax.numpy as jnp
from jax import lax
from jax.experimental import pallas as pl
from jax.experimental.pallas import tpu as pltpu
from jax.experimental.pallas import tpu_sc as plsc

N = 10000
D = 256
C = 7
L = 3
E = 160000
H = 128
P = 8192
BT = 4096
OUT = 8
HALF = 128

NC = 2
NS = 16
RPT = 632
RPT_LAST = N - (NS - 1) * RPT
CH = 128
SLOTS = 3
NCHT = 81
EPT = NCHT * CH
E_PAD = NS * EPT
GROUPS = NCHT // SLOTS

NBLK = 400
GRID = N // NBLK
BB = 512
BGRID = BT // BB

f32 = jnp.float32
i32 = jnp.int32

_mesh = plsc.VectorSubcoreMesh(
    core_axis_name="c", subcore_axis_name="s", num_cores=NC, num_subcores=NS)


def _slabbed(s, fn):
    pl.when(s < NS - 1)(lambda: fn(RPT))
    pl.when(s == NS - 1)(lambda: fn(RPT_LAST))


def _sc_degcnt_body(srcs, dsts, ones_h, z16, degt, cntt,
                    acc, ones_v, idxs, sems):
    c = lax.axis_index("c")
    s = lax.axis_index("s")
    row0 = s * RPT
    _slabbed(s, lambda nr: pltpu.sync_copy(
        z16.at[pl.ds(0, nr)], acc.at[pl.ds(row0, nr)]))
    pltpu.sync_copy(ones_h, ones_v)

    def run(idx_hbm, out):
        plsc.subcore_barrier()

        def group(g, carry):
            pltpu.sync_copy(idx_hbm.at[s, pl.ds(g * SLOTS, SLOTS)], idxs)
            sds = [pltpu.async_copy(
                ones_v, acc.at[idxs.at[b, 0]], sems[b], add=True)
                for b in range(SLOTS)]
            for d in sds:
                d.wait()
            return carry

        lax.fori_loop(0, GROUPS, group, 0)
        plsc.subcore_barrier()
        _slabbed(s, lambda nr: pltpu.sync_copy(
            acc.at[pl.ds(row0, nr)], out.at[pl.ds(row0, nr)]))

    pl.when(c == 0)(lambda: run(dsts, degt))
    pl.when(c == 1)(lambda: run(srcs, cntt))


_sc_degcnt = pl.kernel(
    _sc_degcnt_body,
    out_type=(jax.ShapeDtypeStruct((N, 16), f32),
              jax.ShapeDtypeStruct((N, 16), f32)),
    mesh=_mesh,
    scratch_types=(pltpu.VMEM_SHARED((N + 8, 16), f32),
                   pltpu.VMEM((CH, 16), f32),
                   pltpu.VMEM((SLOTS, 1, CH), i32),
                   tuple(pltpu.SemaphoreType.DMA for _ in range(SLOTS))),
)


def _sc_agg_body(hsf, hf, srcg, dstg, srcs, dsts, zrows, uf, gf,
                 acc, bufs, gidx, sidx, gsems, ssems):
    c = lax.axis_index("c")
    s = lax.axis_index("s")
    row0 = s * RPT
    crow = c * N

    for table, gl, sl, out in ((hsf, srcg, dsts, uf), (hf, dstg, srcs, gf)):
        _slabbed(s, lambda nr: pltpu.sync_copy(
            zrows.at[pl.ds(0, nr)], acc.at[pl.ds(row0, nr)]))
        plsc.subcore_barrier()

        def group(g, carry, table=table, gl=gl, sl=sl):
            pltpu.sync_copy(gl.at[c, s, pl.ds(g * SLOTS, SLOTS)], gidx)
            pltpu.sync_copy(sl.at[s, pl.ds(g * SLOTS, SLOTS)], sidx)
            for b in range(SLOTS):
                pltpu.async_copy(
                    table.at[gidx.at[b, 0]], bufs.at[0], gsems[0]).wait()
                pltpu.sync_copy(bufs.at[0], acc.at[sidx.at[b, 0]], add=True)
            return carry

        lax.fori_loop(0, GROUPS, group, 0)
        plsc.subcore_barrier()
        _slabbed(s, lambda nr, out=out: pltpu.sync_copy(
            acc.at[pl.ds(row0, nr)], out.at[pl.ds(crow + row0, nr)]))
        plsc.subcore_barrier()


_sc_agg = pl.kernel(
    _sc_agg_body,
    out_type=(jax.ShapeDtypeStruct((2 * N, HALF), f32),
              jax.ShapeDtypeStruct((2 * N, HALF), f32)),
    mesh=_mesh,
    scratch_types=(pltpu.VMEM_SHARED((N + 8, HALF), f32),
                   pltpu.VMEM((SLOTS, CH, HALF), f32),
                   pltpu.VMEM((SLOTS, 1, CH), i32),
                   pltpu.VMEM((SLOTS, 1, CH), i32),
                   tuple(pltpu.SemaphoreType.DMA for _ in range(SLOTS)),
                   tuple(pltpu.SemaphoreType.DMA for _ in range(SLOTS))),
)


def _sc_cls_body(pair0, pair1, eid, y0, y1, y2, xa, xb,
                 gbuf, eidb, n0b, n1b, sem):
    c = lax.axis_index("c")
    s = lax.axis_index("s")
    wid = s * NC + c
    base = wid * (BT // (NC * NS))
    pltpu.sync_copy(eid.at[pl.ds(base, 128)], eidb.at[0])
    pltpu.async_copy(pair0.at[eidb.at[0]], n0b, sem).wait()
    pltpu.async_copy(pair1.at[eidb.at[0]], n1b, sem).wait()
    for j, yt in enumerate((y0, y1, y2)):
        pltpu.async_copy(yt.at[n0b], gbuf, sem).wait()
        pltpu.sync_copy(gbuf, xa.at[j, pl.ds(base, 128)])
        pltpu.async_copy(yt.at[n1b], gbuf, sem).wait()
        pltpu.sync_copy(gbuf, xb.at[j, pl.ds(base, 128)])


_sc_cls = pl.kernel(
    _sc_cls_body,
    out_type=(jax.ShapeDtypeStruct((L, BT, H), f32),
              jax.ShapeDtypeStruct((L, BT, H), f32)),
    mesh=_mesh,
    scratch_types=(pltpu.VMEM((128, H), f32),
                   pltpu.VMEM((1, 128), i32),
                   pltpu.VMEM((128,), i32),
                   pltpu.VMEM((128,), i32),
                   pltpu.SemaphoreType.DMA),
)


def _tc_prep_body(x2_r, deg_r, out_r):
    dinv = lax.rsqrt(deg_r[:, :1] + 1.0)
    out_r[0] = x2_r[0] * dinv
    out_r[1] = x2_r[1] * dinv


_tc_prep = pl.pallas_call(
    _tc_prep_body,
    grid=(GRID,),
    in_specs=[pl.BlockSpec((2, NBLK, HALF), lambda r: (0, r, 0)),
              pl.BlockSpec((NBLK, 16), lambda r: (r, 0))],
    out_specs=pl.BlockSpec((2, NBLK, HALF), lambda r: (0, r, 0)),
    out_shape=jax.ShapeDtypeStruct((2, N, HALF), f32),
)


def _tc_dense_body(h2_r, u2_r, g2_r, deg_r, cnt_r,
                   wamp_r, wdg_r, wdf_r, wg1_r, wg2t_r, wlin_r,
                   bamp_r, bdg_r, bdf_r, bg1_r, bg2_r, blin_r,
                   tmp2_r, hs2_r, y_r, st_r):
    r = pl.program_id(0)
    hb = jnp.concatenate([h2_r[0], h2_r[1]], axis=1)
    ub = jnp.concatenate([u2_r[0], u2_r[1]], axis=1)
    gb = jnp.concatenate([g2_r[0], g2_r[1]], axis=1)
    dinv = lax.rsqrt(deg_r[:, :1] + 1.0)
    cnt = cnt_r[:, :1]
    ht = dinv * (ub + dinv * hb)
    dot = functools.partial(jnp.dot, preferred_element_type=f32)
    ha = dot(ht, wamp_r[...]) + bamp_r[...]
    g = jax.nn.relu(dot(ht, wdg_r[...]) + bdg_r[...])
    hd = g + dot(hb * gb, wdf_r[...]) + cnt * bdf_r[...]
    wg1 = wg1_r[...]
    z1 = jax.nn.relu(dot(ha, wg1[:D]) + dot(hd, wg1[D:]) + bg1_r[...])
    a = jax.nn.sigmoid(
        jnp.sum(z1 * wg2t_r[...], axis=1, keepdims=True) + bg2_r[...])
    tmp = a * ha + (1.0 - a) * hd
    hsn = dinv * tmp
    tmp2_r[0] = tmp[:, :HALF]
    tmp2_r[1] = tmp[:, HALF:]
    hs2_r[0] = hsn[:, :HALF]
    hs2_r[1] = hsn[:, HALF:]
    y = dot(tmp, wlin_r[...]) + blin_r[...]
    y_r[...] = y
    part = jnp.concatenate(
        [jnp.sum(y, axis=0, keepdims=True),
         jnp.sum(y * y, axis=0, keepdims=True),
         jnp.zeros((6, H), f32)], axis=0)

    @pl.when(r == 0)
    def _():
        st_r[...] = part

    @pl.when(r > 0)
    def _():
        st_r[...] = st_r[...] + part


_const = lambda shape: pl.BlockSpec(shape, lambda r: tuple(0 for _ in shape))
_tc_dense = pl.pallas_call(
    _tc_dense_body,
    grid=(GRID,),
    in_specs=[pl.BlockSpec((2, NBLK, HALF), lambda r: (0, r, 0)),
              pl.BlockSpec((2, NBLK, HALF), lambda r: (0, r, 0)),
              pl.BlockSpec((2, NBLK, HALF), lambda r: (0, r, 0)),
              pl.BlockSpec((NBLK, 16), lambda r: (r, 0)),
              pl.BlockSpec((NBLK, 16), lambda r: (r, 0)),
              _const((D, D)), _const((D, D)), _const((D, D)),
              _const((2 * D, D)), _const((1, D)), _const((D, H)),
              _const((1, D)), _const((1, D)), _const((1, D)),
              _const((1, D)), _const((1, 1)), _const((1, H))],
    out_specs=[pl.BlockSpec((2, NBLK, HALF), lambda r: (0, r, 0)),
               pl.BlockSpec((2, NBLK, HALF), lambda r: (0, r, 0)),
               pl.BlockSpec((NBLK, H), lambda r: (r, 0)),
               pl.BlockSpec((8, H), lambda r: (0, 0))],
    out_shape=[jax.ShapeDtypeStruct((2, N, HALF), f32),
               jax.ShapeDtypeStruct((2, N, HALF), f32),
               jax.ShapeDtypeStruct((N, H), f32),
               jax.ShapeDtypeStruct((8, H), f32)],
)


def _tc_cls_body(xa_r, xb_r, st0_r, st1_r, st2_r, gam_r, bet_r,
                 wf1_r, bf1_r, wf2_r, bf2_r, wf3_r, bf3_r, o_r):
    x1s, x2s = [], []
    for j, st_r in enumerate((st0_r, st1_r, st2_r)):
        st = st_r[...]
        mu = st[0:1] / N
        var = st[1:2] / N - mu * mu
        sc = gam_r[j:j + 1] * lax.rsqrt(var + 1e-5)
        bt = bet_r[j:j + 1]
        x1s.append(jax.nn.relu(sc * (xa_r[j] - mu) + bt))
        x2s.append(jax.nn.relu(sc * (xb_r[j] - mu) + bt))
    x1 = jnp.concatenate(x1s, axis=1)
    x2 = jnp.concatenate(x2s, axis=1)
    t = jnp.concatenate([x1, x2, x1 * x2], axis=1)
    dot = functools.partial(jnp.dot, preferred_element_type=f32)
    t1 = jax.nn.relu(dot(t, wf1_r[...]) + bf1_r[...])
    t2 = jax.nn.relu(dot(t1, wf2_r[...]) + bf2_r[...])
    o_r[...] = dot(t2, wf3_r[...]) + bf3_r[...]


F1 = 3 * H * L
F2 = F1 // 2
F3 = F1 // 4
_tc_cls = pl.pallas_call(
    _tc_cls_body,
    grid=(BGRID,),
    in_specs=[pl.BlockSpec((L, BB, H), lambda r: (0, r, 0)),
              pl.BlockSpec((L, BB, H), lambda r: (0, r, 0)),
              _const((8, H)), _const((8, H)), _const((8, H)),
              _const((L, H)), _const((L, H)),
              _const((F1, F2)), _const((1, F2)),
              _const((F2, F3)), _const((1, F3)),
              _const((F3, OUT)), _const((1, OUT))],
    out_specs=pl.BlockSpec((BB, OUT), lambda r: (r, 0)),
    out_shape=jax.ShapeDtypeStruct((BT, OUT), f32),
)


def _tc_final_body(r_r, wc_r, bc_r, o_r):
    o_r[...] = jnp.dot(jax.nn.relu(r_r[...]), wc_r[...],
                       preferred_element_type=f32) + bc_r[...]


_tc_final = pl.pallas_call(
    _tc_final_body,
    out_shape=jax.ShapeDtypeStruct((BT, C), f32),
)


def kernel(x, edges, pair_index, edge_id, W_amp, b_amp, W_dgcn, b_dgcn,
           W_diff, b_diff, Wg1, bg1, Wg2, bg2, W_lin, b_lin, gamma, beta,
           W_fc1, b_fc1, W_fc2, b_fc2, W_fc3, b_fc3, W_cls, b_cls):
    x = x.astype(f32)
    x2 = jnp.transpose(jnp.reshape(x, (N, 2, HALF)), (1, 0, 2))
    xf = jnp.reshape(x2, (2 * N, HALF))
    zrows = jnp.zeros((RPT, HALF), f32)
    z16 = jnp.zeros((RPT, 16), f32)
    ones16 = jnp.ones((CH, 16), f32)
    pair0 = pair_index[0].astype(i32)
    pair1 = pair_index[1].astype(i32)
    eid = edge_id.astype(i32)

    padg = jnp.zeros((E_PAD - E,), i32)
    pads = jnp.full((E_PAD - E,), N, i32)
    outs = []
    for i in range(C):
        src = edges[i, 0].astype(i32)
        dst = edges[i, 1].astype(i32)
        srcp = jnp.concatenate([src, padg])
        dstp = jnp.concatenate([dst, padg])
        srcg = jnp.stack([srcp, srcp + N]).reshape(2, NS, NCHT, 1, CH)
        dstg = jnp.stack([dstp, dstp + N]).reshape(2, NS, NCHT, 1, CH)
        srcs = jnp.concatenate([src, pads]).reshape(NS, NCHT, 1, CH)
        dsts = jnp.concatenate([dst, pads]).reshape(NS, NCHT, 1, CH)
        degt, cntt = _sc_degcnt(srcs, dsts, ones16, z16)
        h2 = x2
        hf = xf
        hsf = jnp.reshape(_tc_prep(x2, degt), (2 * N, HALF))
        ys, sts = [], []
        for j in range(L):
            uf, gf = _sc_agg(hsf, hf, srcg, dstg, srcs, dsts, zrows)
            tmp2, hs2, y, st = _tc_dense(
                h2, jnp.reshape(uf, (2, N, HALF)), jnp.reshape(gf, (2, N, HALF)),
                degt, cntt,
                W_amp[i, j], W_dgcn[i, j], W_diff[i, j], Wg1[i, j],
                jnp.reshape(Wg2[i, j], (1, D)), W_lin[i, j],
                jnp.reshape(b_amp[i, j], (1, D)), jnp.reshape(b_dgcn[i, j], (1, D)),
                jnp.reshape(b_diff[i, j], (1, D)), jnp.reshape(bg1[i, j], (1, D)),
                jnp.reshape(bg2[i, j], (1, 1)), jnp.reshape(b_lin[i, j], (1, H)))
            h2 = tmp2
            hf = jnp.reshape(tmp2, (2 * N, HALF))
            hsf = jnp.reshape(hs2, (2 * N, HALF))
            ys.append(y)
            sts.append(st)
        xa, xb = _sc_cls(pair0, pair1, eid, ys[0], ys[1], ys[2])
        o = _tc_cls(xa, xb, sts[0], sts[1], sts[2], gamma[i], beta[i],
                    W_fc1[i], jnp.reshape(b_fc1[i], (1, F2)),
                    W_fc2[i], jnp.reshape(b_fc2[i], (1, F3)),
                    W_fc3[i], jnp.reshape(b_fc3[i], (1, OUT)))
        outs.append(o)
    rcat = jnp.concatenate(outs, axis=1)
    return _tc_final(rcat, W_cls, jnp.reshape(b_cls, (1, C)))


# speedup vs baseline: 1.1178x; 1.1178x over previous
import functools

import jax
import jax.numpy as jnp
from jax import lax
from jax.experimental import pallas as pl
from jax.experimental.pallas import tpu as pltpu
from jax.experimental.pallas import tpu_sc as plsc

N = 10000
D = 256
C = 7
L = 3
E = 160000
H = 128
P = 8192
BT = 4096
OUT = 8
HALF = 128

NC = 2
NS = 16
RPT = 632
RPT_LAST = N - (NS - 1) * RPT
CH = 128
SLOTS = 2
NCHT = 80
EPT = NCHT * CH
E_PAD = NS * EPT
GROUPS = NCHT // SLOTS

NBLK = 400
GRID = N // NBLK
BB = 512
BGRID = BT // BB

f32 = jnp.float32
i32 = jnp.int32

_mesh = plsc.VectorSubcoreMesh(
    core_axis_name="c", subcore_axis_name="s", num_cores=NC, num_subcores=NS)


def _slabbed(s, fn):
    pl.when(s < NS - 1)(lambda: fn(RPT))
    pl.when(s == NS - 1)(lambda: fn(RPT_LAST))


def _sc_degcnt_body(srcs, dsts, ones_h, z16, degt, cntt,
                    acc, ones_v, idxs, sems):
    c = lax.axis_index("c")
    s = lax.axis_index("s")
    row0 = s * RPT
    ebase = s * EPT
    _slabbed(s, lambda nr: pltpu.sync_copy(
        z16.at[pl.ds(0, nr)], acc.at[pl.ds(row0, nr)]))
    pltpu.sync_copy(ones_h, ones_v)

    def run(idx_hbm, out):
        plsc.subcore_barrier()

        def step(k, carry):
            pltpu.sync_copy(idx_hbm.at[pl.ds(ebase + k * CH, CH)], idxs.at[0])
            pltpu.sync_copy(ones_v, acc.at[idxs.at[0]], add=True)
            return carry

        lax.fori_loop(0, NCHT, step, 0)
        plsc.subcore_barrier()
        _slabbed(s, lambda nr: pltpu.sync_copy(
            acc.at[pl.ds(row0, nr)], out.at[pl.ds(row0, nr)]))

    pl.when(c == 0)(lambda: run(dsts, degt))
    pl.when(c == 1)(lambda: run(srcs, cntt))


_sc_degcnt = pl.kernel(
    _sc_degcnt_body,
    out_type=(jax.ShapeDtypeStruct((N, 16), f32),
              jax.ShapeDtypeStruct((N, 16), f32)),
    mesh=_mesh,
    scratch_types=(pltpu.VMEM_SHARED((N + 8, 16), f32),
                   pltpu.VMEM((CH, 16), f32),
                   pltpu.VMEM((1, CH), i32),
                   tuple(pltpu.SemaphoreType.DMA for _ in range(SLOTS))),
)


def _sc_agg_body(hsf, hf, srcg, dstg, srcs, dsts, zrows, uf, gf,
                 acc, bufs, gidx, sidx, gsems, ssems):
    c = lax.axis_index("c")
    s = lax.axis_index("s")
    row0 = s * RPT
    crow = c * N
    ebase = s * EPT
    cE = c * E_PAD

    for table, gl, sl, out in ((hsf, srcg, dsts, uf), (hf, dstg, srcs, gf)):
        _slabbed(s, lambda nr: pltpu.sync_copy(
            zrows.at[pl.ds(0, nr)], acc.at[pl.ds(row0, nr)]))
        plsc.subcore_barrier()

        def group(g, carry, table=table, gl=gl, sl=sl):
            for b in range(SLOTS):
                off = ebase + (g * SLOTS + b) * CH
                pltpu.sync_copy(gl.at[pl.ds(cE + off, CH)], gidx.at[b])
                pltpu.sync_copy(sl.at[pl.ds(off, CH)], sidx.at[b])
            gds = [pltpu.async_copy(
                table.at[gidx.at[b]], bufs.at[b], gsems[b])
                for b in range(SLOTS)]
            sds = []
            for b in range(SLOTS):
                gds[b].wait()
                sds.append(pltpu.async_copy(
                    bufs.at[b], acc.at[sidx.at[b]], ssems[b], add=True))
            for d in sds:
                d.wait()
            return carry

        lax.fori_loop(0, GROUPS, group, 0)
        plsc.subcore_barrier()
        _slabbed(s, lambda nr, out=out: pltpu.sync_copy(
            acc.at[pl.ds(row0, nr)], out.at[pl.ds(crow + row0, nr)]))
        plsc.subcore_barrier()


_sc_agg = pl.kernel(
    _sc_agg_body,
    out_type=(jax.ShapeDtypeStruct((2 * N, HALF), f32),
              jax.ShapeDtypeStruct((2 * N, HALF), f32)),
    mesh=_mesh,
    scratch_types=(pltpu.VMEM_SHARED((N + 8, HALF), f32),
                   pltpu.VMEM((SLOTS, CH, HALF), f32),
                   pltpu.VMEM((SLOTS, CH), i32),
                   pltpu.VMEM((SLOTS, CH), i32),
                   tuple(pltpu.SemaphoreType.DMA for _ in range(SLOTS)),
                   tuple(pltpu.SemaphoreType.DMA for _ in range(SLOTS))),
)


def _sc_cls_body(pair0, pair1, eid, y0, y1, y2, xa, xb,
                 gbuf, eidb, n0b, n1b, sem):
    c = lax.axis_index("c")
    s = lax.axis_index("s")
    wid = s * NC + c
    base = wid * (BT // (NC * NS))
    pltpu.sync_copy(eid.at[pl.ds(base, 128)], eidb.at[0])
    pltpu.async_copy(pair0.at[eidb.at[0]], n0b, sem).wait()
    pltpu.async_copy(pair1.at[eidb.at[0]], n1b, sem).wait()
    for j, yt in enumerate((y0, y1, y2)):
        pltpu.async_copy(yt.at[n0b], gbuf, sem).wait()
        pltpu.sync_copy(gbuf, xa.at[j, pl.ds(base, 128)])
        pltpu.async_copy(yt.at[n1b], gbuf, sem).wait()
        pltpu.sync_copy(gbuf, xb.at[j, pl.ds(base, 128)])


_sc_cls = pl.kernel(
    _sc_cls_body,
    out_type=(jax.ShapeDtypeStruct((L, BT, H), f32),
              jax.ShapeDtypeStruct((L, BT, H), f32)),
    mesh=_mesh,
    scratch_types=(pltpu.VMEM((128, H), f32),
                   pltpu.VMEM((1, 128), i32),
                   pltpu.VMEM((128,), i32),
                   pltpu.VMEM((128,), i32),
                   pltpu.SemaphoreType.DMA),
)


def _tc_prep_body(x2_r, deg_r, out_r):
    dinv = lax.rsqrt(deg_r[:, :1] + 1.0)
    out_r[0] = x2_r[0] * dinv
    out_r[1] = x2_r[1] * dinv


_tc_prep = pl.pallas_call(
    _tc_prep_body,
    grid=(GRID,),
    in_specs=[pl.BlockSpec((2, NBLK, HALF), lambda r: (0, r, 0)),
              pl.BlockSpec((NBLK, 16), lambda r: (r, 0))],
    out_specs=pl.BlockSpec((2, NBLK, HALF), lambda r: (0, r, 0)),
    out_shape=jax.ShapeDtypeStruct((2, N, HALF), f32),
)


def _tc_dense_body(h2_r, u2_r, g2_r, deg_r, cnt_r,
                   wamp_r, wdg_r, wdf_r, wg1_r, wg2t_r, wlin_r,
                   bamp_r, bdg_r, bdf_r, bg1_r, bg2_r, blin_r,
                   tmp2_r, hs2_r, y_r, st_r):
    r = pl.program_id(0)
    hb = jnp.concatenate([h2_r[0], h2_r[1]], axis=1)
    ub = jnp.concatenate([u2_r[0], u2_r[1]], axis=1)
    gb = jnp.concatenate([g2_r[0], g2_r[1]], axis=1)
    dinv = lax.rsqrt(deg_r[:, :1] + 1.0)
    cnt = cnt_r[:, :1]
    ht = dinv * (ub + dinv * hb)
    dot = functools.partial(jnp.dot, preferred_element_type=f32)
    ha = dot(ht, wamp_r[...]) + bamp_r[...]
    g = jax.nn.relu(dot(ht, wdg_r[...]) + bdg_r[...])
    hd = g + dot(hb * gb, wdf_r[...]) + cnt * bdf_r[...]
    wg1 = wg1_r[...]
    z1 = jax.nn.relu(dot(ha, wg1[:D]) + dot(hd, wg1[D:]) + bg1_r[...])
    a = jax.nn.sigmoid(
        jnp.sum(z1 * wg2t_r[...], axis=1, keepdims=True) + bg2_r[...])
    tmp = a * ha + (1.0 - a) * hd
    hsn = dinv * tmp
    tmp2_r[0] = tmp[:, :HALF]
    tmp2_r[1] = tmp[:, HALF:]
    hs2_r[0] = hsn[:, :HALF]
    hs2_r[1] = hsn[:, HALF:]
    y = dot(tmp, wlin_r[...]) + blin_r[...]
    y_r[...] = y
    part = jnp.concatenate(
        [jnp.sum(y, axis=0, keepdims=True),
         jnp.sum(y * y, axis=0, keepdims=True),
         jnp.zeros((6, H), f32)], axis=0)

    @pl.when(r == 0)
    def _():
        st_r[...] = part

    @pl.when(r > 0)
    def _():
        st_r[...] = st_r[...] + part


_const = lambda shape: pl.BlockSpec(shape, lambda r: tuple(0 for _ in shape))
_tc_dense = pl.pallas_call(
    _tc_dense_body,
    grid=(GRID,),
    in_specs=[pl.BlockSpec((2, NBLK, HALF), lambda r: (0, r, 0)),
              pl.BlockSpec((2, NBLK, HALF), lambda r: (0, r, 0)),
              pl.BlockSpec((2, NBLK, HALF), lambda r: (0, r, 0)),
              pl.BlockSpec((NBLK, 16), lambda r: (r, 0)),
              pl.BlockSpec((NBLK, 16), lambda r: (r, 0)),
              _const((D, D)), _const((D, D)), _const((D, D)),
              _const((2 * D, D)), _const((1, D)), _const((D, H)),
              _const((1, D)), _const((1, D)), _const((1, D)),
              _const((1, D)), _const((1, 1)), _const((1, H))],
    out_specs=[pl.BlockSpec((2, NBLK, HALF), lambda r: (0, r, 0)),
               pl.BlockSpec((2, NBLK, HALF), lambda r: (0, r, 0)),
               pl.BlockSpec((NBLK, H), lambda r: (r, 0)),
               pl.BlockSpec((8, H), lambda r: (0, 0))],
    out_shape=[jax.ShapeDtypeStruct((2, N, HALF), f32),
               jax.ShapeDtypeStruct((2, N, HALF), f32),
               jax.ShapeDtypeStruct((N, H), f32),
               jax.ShapeDtypeStruct((8, H), f32)],
)


def _tc_cls_body(xa_r, xb_r, st0_r, st1_r, st2_r, gam_r, bet_r,
                 wf1_r, bf1_r, wf2_r, bf2_r, wf3_r, bf3_r, o_r):
    x1s, x2s = [], []
    for j, st_r in enumerate((st0_r, st1_r, st2_r)):
        st = st_r[...]
        mu = st[0:1] / N
        var = st[1:2] / N - mu * mu
        sc = gam_r[j:j + 1] * lax.rsqrt(var + 1e-5)
        bt = bet_r[j:j + 1]
        x1s.append(jax.nn.relu(sc * (xa_r[j] - mu) + bt))
        x2s.append(jax.nn.relu(sc * (xb_r[j] - mu) + bt))
    x1 = jnp.concatenate(x1s, axis=1)
    x2 = jnp.concatenate(x2s, axis=1)
    t = jnp.concatenate([x1, x2, x1 * x2], axis=1)
    dot = functools.partial(jnp.dot, preferred_element_type=f32)
    t1 = jax.nn.relu(dot(t, wf1_r[...]) + bf1_r[...])
    t2 = jax.nn.relu(dot(t1, wf2_r[...]) + bf2_r[...])
    o_r[...] = dot(t2, wf3_r[...]) + bf3_r[...]


F1 = 3 * H * L
F2 = F1 // 2
F3 = F1 // 4
_tc_cls = pl.pallas_call(
    _tc_cls_body,
    grid=(BGRID,),
    in_specs=[pl.BlockSpec((L, BB, H), lambda r: (0, r, 0)),
              pl.BlockSpec((L, BB, H), lambda r: (0, r, 0)),
              _const((8, H)), _const((8, H)), _const((8, H)),
              _const((L, H)), _const((L, H)),
              _const((F1, F2)), _const((1, F2)),
              _const((F2, F3)), _const((1, F3)),
              _const((F3, OUT)), _const((1, OUT))],
    out_specs=pl.BlockSpec((BB, OUT), lambda r: (r, 0)),
    out_shape=jax.ShapeDtypeStruct((BT, OUT), f32),
)


def _tc_final_body(r_r, wc_r, bc_r, o_r):
    o_r[...] = jnp.dot(jax.nn.relu(r_r[...]), wc_r[...],
                       preferred_element_type=f32) + bc_r[...]


_tc_final = pl.pallas_call(
    _tc_final_body,
    out_shape=jax.ShapeDtypeStruct((BT, C), f32),
)


def kernel(x, edges, pair_index, edge_id, W_amp, b_amp, W_dgcn, b_dgcn,
           W_diff, b_diff, Wg1, bg1, Wg2, bg2, W_lin, b_lin, gamma, beta,
           W_fc1, b_fc1, W_fc2, b_fc2, W_fc3, b_fc3, W_cls, b_cls):
    x = x.astype(f32)
    x2 = jnp.transpose(jnp.reshape(x, (N, 2, HALF)), (1, 0, 2))
    xf = jnp.reshape(x2, (2 * N, HALF))
    zrows = jnp.zeros((RPT, HALF), f32)
    z16 = jnp.zeros((RPT, 16), f32)
    ones16 = jnp.ones((CH, 16), f32)
    pair0 = pair_index[0].astype(i32)
    pair1 = pair_index[1].astype(i32)
    eid = edge_id.astype(i32)

    padg = jnp.zeros((E_PAD - E,), i32)
    pads = jnp.full((E_PAD - E,), N, i32)
    outs = []
    for i in range(C):
        src = edges[i, 0].astype(i32)
        dst = edges[i, 1].astype(i32)
        srcp = jnp.concatenate([src, padg])
        dstp = jnp.concatenate([dst, padg])
        srcg = jnp.concatenate([srcp, srcp + N])
        dstg = jnp.concatenate([dstp, dstp + N])
        srcs = jnp.concatenate([src, pads])
        dsts = jnp.concatenate([dst, pads])
        degt, cntt = _sc_degcnt(srcs, dsts, ones16, z16)
        h2 = x2
        hf = xf
        hsf = jnp.reshape(_tc_prep(x2, degt), (2 * N, HALF))
        ys, sts = [], []
        for j in range(L):
            uf, gf = _sc_agg(hsf, hf, srcg, dstg, srcs, dsts, zrows)
            tmp2, hs2, y, st = _tc_dense(
                h2, jnp.reshape(uf, (2, N, HALF)), jnp.reshape(gf, (2, N, HALF)),
                degt, cntt,
                W_amp[i, j], W_dgcn[i, j], W_diff[i, j], Wg1[i, j],
                jnp.reshape(Wg2[i, j], (1, D)), W_lin[i, j],
                jnp.reshape(b_amp[i, j], (1, D)), jnp.reshape(b_dgcn[i, j], (1, D)),
                jnp.reshape(b_diff[i, j], (1, D)), jnp.reshape(bg1[i, j], (1, D)),
                jnp.reshape(bg2[i, j], (1, 1)), jnp.reshape(b_lin[i, j], (1, H)))
            h2 = tmp2
            hf = jnp.reshape(tmp2, (2 * N, HALF))
            hsf = jnp.reshape(hs2, (2 * N, HALF))
            ys.append(y)
            sts.append(st)
        xa, xb = _sc_cls(pair0, pair1, eid, ys[0], ys[1], ys[2])
        o = _tc_cls(xa, xb, sts[0], sts[1], sts[2], gamma[i], beta[i],
                    W_fc1[i], jnp.reshape(b_fc1[i], (1, F2)),
                    W_fc2[i], jnp.reshape(b_fc2[i], (1, F3)),
                    W_fc3[i], jnp.reshape(b_fc3[i], (1, OUT)))
        outs.append(o)
    rcat = jnp.concatenate(outs, axis=1)
    return _tc_final(rcat, W_cls, jnp.reshape(b_cls, (1, C)))


# speedup vs baseline: 2.0166x; 1.8042x over previous
import functools

import jax
import jax.numpy as jnp
from jax import lax
from jax.experimental import pallas as pl
from jax.experimental.pallas import tpu as pltpu
from jax.experimental.pallas import tpu_sc as plsc

N = 10000
D = 256
C = 7
L = 3
E = 160000
H = 128
P = 8192
BT = 4096
OUT = 8
HALF = 128

NC = 2
NS = 16
RPT = 632
RPT_LAST = N - (NS - 1) * RPT
CH = 128
SLOTS = 2
EPT = E // NS
NCHT = EPT // CH
TAIL = EPT - NCHT * CH
GROUPS = NCHT // SLOTS

NBLK = 400
GRID = N // NBLK
BB = 512
BGRID = BT // BB

f32 = jnp.float32
i32 = jnp.int32

_mesh = plsc.VectorSubcoreMesh(
    core_axis_name="c", subcore_axis_name="s", num_cores=NC, num_subcores=NS)


def _slabbed(s, fn):
    pl.when(s < NS - 1)(lambda: fn(RPT))
    pl.when(s == NS - 1)(lambda: fn(RPT_LAST))


def _sc_degcnt_body(srcs, dsts, ones_h, z16, degt, cntt,
                    acc, ones_v, idxs, idxt, sems):
    c = lax.axis_index("c")
    s = lax.axis_index("s")
    row0 = s * RPT
    ebase = s * EPT
    _slabbed(s, lambda nr: pltpu.sync_copy(
        z16.at[pl.ds(0, nr)], acc.at[pl.ds(row0, nr)]))
    pltpu.sync_copy(ones_h, ones_v)

    def run(idx_hbm, out):
        plsc.subcore_barrier()

        def step(k, carry):
            pltpu.sync_copy(idx_hbm.at[pl.ds(ebase + k * CH, CH)], idxs.at[0])
            pltpu.sync_copy(ones_v, acc.at[idxs.at[0]], add=True)
            return carry

        lax.fori_loop(0, NCHT, step, 0)
        pltpu.sync_copy(idx_hbm.at[pl.ds(ebase + NCHT * CH, TAIL)], idxt.at[0])
        pltpu.sync_copy(ones_v.at[pl.ds(0, TAIL)], acc.at[idxt.at[0]], add=True)
        plsc.subcore_barrier()
        _slabbed(s, lambda nr: pltpu.sync_copy(
            acc.at[pl.ds(row0, nr)], out.at[pl.ds(row0, nr)]))

    pl.when(c == 0)(lambda: run(dsts, degt))
    pl.when(c == 1)(lambda: run(srcs, cntt))


_sc_degcnt = pl.kernel(
    _sc_degcnt_body,
    out_type=(jax.ShapeDtypeStruct((N, 16), f32),
              jax.ShapeDtypeStruct((N, 16), f32)),
    mesh=_mesh,
    scratch_types=(pltpu.VMEM_SHARED((N, 16), f32),
                   pltpu.VMEM((CH, 16), f32),
                   pltpu.VMEM((1, CH), i32),
                   pltpu.VMEM((1, TAIL), i32),
                   tuple(pltpu.SemaphoreType.DMA for _ in range(SLOTS))),
)


def _sc_agg_body(hsf, hf, srcg, dstg, srcs, dsts, zrows, uf, gf,
                 acc, bufs, gidx0, gidx1, sidx0, sidx1, gidxt, sidxt,
                 gsems, ssems):
    c = lax.axis_index("c")
    s = lax.axis_index("s")
    row0 = s * RPT
    crow = c * N
    ebase = s * EPT
    cE = c * E
    gidxs = (gidx0, gidx1)
    sidxs = (sidx0, sidx1)

    for table, gl, sl, out in ((hsf, srcg, dsts, uf), (hf, dstg, srcs, gf)):
        _slabbed(s, lambda nr: pltpu.sync_copy(
            zrows.at[pl.ds(0, nr)], acc.at[pl.ds(row0, nr)]))
        plsc.subcore_barrier()

        def group(g, carry, table=table, gl=gl, sl=sl):
            for b in range(SLOTS):
                off = ebase + (g * SLOTS + b) * CH
                pltpu.sync_copy(gl.at[pl.ds(cE + off, CH)], gidxs[b].at[0])
                pltpu.sync_copy(sl.at[pl.ds(off, CH)], sidxs[b].at[0])
            gds = [pltpu.async_copy(
                table.at[gidxs[b].at[0]], bufs.at[b], gsems[b])
                for b in range(SLOTS)]
            sds = []
            for b in range(SLOTS):
                gds[b].wait()
                sds.append(pltpu.async_copy(
                    bufs.at[b], acc.at[sidxs[b].at[0]], ssems[b], add=True))
            for d in sds:
                d.wait()
            return carry

        lax.fori_loop(0, GROUPS, group, 0)
        offt = ebase + NCHT * CH
        pltpu.sync_copy(gl.at[pl.ds(cE + offt, TAIL)], gidxt.at[0])
        pltpu.sync_copy(sl.at[pl.ds(offt, TAIL)], sidxt.at[0])
        pltpu.async_copy(
            table.at[gidxt.at[0]], bufs.at[0, pl.ds(0, TAIL)], gsems[0]).wait()
        pltpu.sync_copy(bufs.at[0, pl.ds(0, TAIL)], acc.at[sidxt.at[0]],
                        add=True)
        plsc.subcore_barrier()
        _slabbed(s, lambda nr, out=out: pltpu.sync_copy(
            acc.at[pl.ds(row0, nr)], out.at[pl.ds(crow + row0, nr)]))
        plsc.subcore_barrier()


_sc_agg = pl.kernel(
    _sc_agg_body,
    out_type=(jax.ShapeDtypeStruct((2 * N, HALF), f32),
              jax.ShapeDtypeStruct((2 * N, HALF), f32)),
    mesh=_mesh,
    scratch_types=(pltpu.VMEM_SHARED((N, HALF), f32),
                   pltpu.VMEM((SLOTS, CH, HALF), f32),
                   pltpu.VMEM((1, CH), i32),
                   pltpu.VMEM((1, CH), i32),
                   pltpu.VMEM((1, CH), i32),
                   pltpu.VMEM((1, CH), i32),
                   pltpu.VMEM((1, TAIL), i32),
                   pltpu.VMEM((1, TAIL), i32),
                   tuple(pltpu.SemaphoreType.DMA for _ in range(SLOTS)),
                   tuple(pltpu.SemaphoreType.DMA for _ in range(SLOTS))),
)


def _sc_cls_body(pair0, pair1, eid, y0, y1, y2, xa, xb,
                 gbuf, eidb, n0b, n1b, sem):
    c = lax.axis_index("c")
    s = lax.axis_index("s")
    wid = s * NC + c
    base = wid * (BT // (NC * NS))
    pltpu.sync_copy(eid.at[pl.ds(base, 128)], eidb.at[0])
    pltpu.async_copy(pair0.at[eidb.at[0]], n0b, sem).wait()
    pltpu.async_copy(pair1.at[eidb.at[0]], n1b, sem).wait()
    for j, yt in enumerate((y0, y1, y2)):
        pltpu.async_copy(yt.at[n0b], gbuf, sem).wait()
        pltpu.sync_copy(gbuf, xa.at[j, pl.ds(base, 128)])
        pltpu.async_copy(yt.at[n1b], gbuf, sem).wait()
        pltpu.sync_copy(gbuf, xb.at[j, pl.ds(base, 128)])


_sc_cls = pl.kernel(
    _sc_cls_body,
    out_type=(jax.ShapeDtypeStruct((L, BT, H), f32),
              jax.ShapeDtypeStruct((L, BT, H), f32)),
    mesh=_mesh,
    scratch_types=(pltpu.VMEM((128, H), f32),
                   pltpu.VMEM((1, 128), i32),
                   pltpu.VMEM((128,), i32),
                   pltpu.VMEM((128,), i32),
                   pltpu.SemaphoreType.DMA),
)


def _tc_prep_body(x2_r, deg_r, out_r):
    dinv = lax.rsqrt(deg_r[:, :1] + 1.0)
    out_r[0] = x2_r[0] * dinv
    out_r[1] = x2_r[1] * dinv


_tc_prep = pl.pallas_call(
    _tc_prep_body,
    grid=(GRID,),
    in_specs=[pl.BlockSpec((2, NBLK, HALF), lambda r: (0, r, 0)),
              pl.BlockSpec((NBLK, 16), lambda r: (r, 0))],
    out_specs=pl.BlockSpec((2, NBLK, HALF), lambda r: (0, r, 0)),
    out_shape=jax.ShapeDtypeStruct((2, N, HALF), f32),
)


def _tc_dense_body(h2_r, u2_r, g2_r, deg_r, cnt_r,
                   wamp_r, wdg_r, wdf_r, wg1_r, wg2t_r, wlin_r,
                   bamp_r, bdg_r, bdf_r, bg1_r, bg2_r, blin_r,
                   tmp2_r, hs2_r, y_r, st_r):
    r = pl.program_id(0)
    hb = jnp.concatenate([h2_r[0], h2_r[1]], axis=1)
    ub = jnp.concatenate([u2_r[0], u2_r[1]], axis=1)
    gb = jnp.concatenate([g2_r[0], g2_r[1]], axis=1)
    dinv = lax.rsqrt(deg_r[:, :1] + 1.0)
    cnt = cnt_r[:, :1]
    ht = dinv * (ub + dinv * hb)
    dot = functools.partial(jnp.dot, preferred_element_type=f32)
    ha = dot(ht, wamp_r[...]) + bamp_r[...]
    g = jax.nn.relu(dot(ht, wdg_r[...]) + bdg_r[...])
    hd = g + dot(hb * gb, wdf_r[...]) + cnt * bdf_r[...]
    wg1 = wg1_r[...]
    z1 = jax.nn.relu(dot(ha, wg1[:D]) + dot(hd, wg1[D:]) + bg1_r[...])
    a = jax.nn.sigmoid(
        jnp.sum(z1 * wg2t_r[...], axis=1, keepdims=True) + bg2_r[...])
    tmp = a * ha + (1.0 - a) * hd
    hsn = dinv * tmp
    tmp2_r[0] = tmp[:, :HALF]
    tmp2_r[1] = tmp[:, HALF:]
    hs2_r[0] = hsn[:, :HALF]
    hs2_r[1] = hsn[:, HALF:]
    y = dot(tmp, wlin_r[...]) + blin_r[...]
    y_r[...] = y
    part = jnp.concatenate(
        [jnp.sum(y, axis=0, keepdims=True),
         jnp.sum(y * y, axis=0, keepdims=True),
         jnp.zeros((6, H), f32)], axis=0)

    @pl.when(r == 0)
    def _():
        st_r[...] = part

    @pl.when(r > 0)
    def _():
        st_r[...] = st_r[...] + part


_const = lambda shape: pl.BlockSpec(shape, lambda r: tuple(0 for _ in shape))
_tc_dense = pl.pallas_call(
    _tc_dense_body,
    grid=(GRID,),
    in_specs=[pl.BlockSpec((2, NBLK, HALF), lambda r: (0, r, 0)),
              pl.BlockSpec((2, NBLK, HALF), lambda r: (0, r, 0)),
              pl.BlockSpec((2, NBLK, HALF), lambda r: (0, r, 0)),
              pl.BlockSpec((NBLK, 16), lambda r: (r, 0)),
              pl.BlockSpec((NBLK, 16), lambda r: (r, 0)),
              _const((D, D)), _const((D, D)), _const((D, D)),
              _const((2 * D, D)), _const((1, D)), _const((D, H)),
              _const((1, D)), _const((1, D)), _const((1, D)),
              _const((1, D)), _const((1, 1)), _const((1, H))],
    out_specs=[pl.BlockSpec((2, NBLK, HALF), lambda r: (0, r, 0)),
               pl.BlockSpec((2, NBLK, HALF), lambda r: (0, r, 0)),
               pl.BlockSpec((NBLK, H), lambda r: (r, 0)),
               pl.BlockSpec((8, H), lambda r: (0, 0))],
    out_shape=[jax.ShapeDtypeStruct((2, N, HALF), f32),
               jax.ShapeDtypeStruct((2, N, HALF), f32),
               jax.ShapeDtypeStruct((N, H), f32),
               jax.ShapeDtypeStruct((8, H), f32)],
)


def _tc_cls_body(xa_r, xb_r, st0_r, st1_r, st2_r, gam_r, bet_r,
                 wf1_r, bf1_r, wf2_r, bf2_r, wf3_r, bf3_r, o_r):
    x1s, x2s = [], []
    for j, st_r in enumerate((st0_r, st1_r, st2_r)):
        st = st_r[...]
        mu = st[0:1] / N
        var = st[1:2] / N - mu * mu
        sc = gam_r[j:j + 1] * lax.rsqrt(var + 1e-5)
        bt = bet_r[j:j + 1]
        x1s.append(jax.nn.relu(sc * (xa_r[j] - mu) + bt))
        x2s.append(jax.nn.relu(sc * (xb_r[j] - mu) + bt))
    x1 = jnp.concatenate(x1s, axis=1)
    x2 = jnp.concatenate(x2s, axis=1)
    t = jnp.concatenate([x1, x2, x1 * x2], axis=1)
    dot = functools.partial(jnp.dot, preferred_element_type=f32)
    t1 = jax.nn.relu(dot(t, wf1_r[...]) + bf1_r[...])
    t2 = jax.nn.relu(dot(t1, wf2_r[...]) + bf2_r[...])
    o_r[...] = dot(t2, wf3_r[...]) + bf3_r[...]


F1 = 3 * H * L
F2 = F1 // 2
F3 = F1 // 4
_tc_cls = pl.pallas_call(
    _tc_cls_body,
    grid=(BGRID,),
    in_specs=[pl.BlockSpec((L, BB, H), lambda r: (0, r, 0)),
              pl.BlockSpec((L, BB, H), lambda r: (0, r, 0)),
              _const((8, H)), _const((8, H)), _const((8, H)),
              _const((L, H)), _const((L, H)),
              _const((F1, F2)), _const((1, F2)),
              _const((F2, F3)), _const((1, F3)),
              _const((F3, OUT)), _const((1, OUT))],
    out_specs=pl.BlockSpec((BB, OUT), lambda r: (r, 0)),
    out_shape=jax.ShapeDtypeStruct((BT, OUT), f32),
)


def _tc_final_body(r_r, wc_r, bc_r, o_r):
    o_r[...] = jnp.dot(jax.nn.relu(r_r[...]), wc_r[...],
                       preferred_element_type=f32) + bc_r[...]


_tc_final = pl.pallas_call(
    _tc_final_body,
    out_shape=jax.ShapeDtypeStruct((BT, C), f32),
)


def kernel(x, edges, pair_index, edge_id, W_amp, b_amp, W_dgcn, b_dgcn,
           W_diff, b_diff, Wg1, bg1, Wg2, bg2, W_lin, b_lin, gamma, beta,
           W_fc1, b_fc1, W_fc2, b_fc2, W_fc3, b_fc3, W_cls, b_cls):
    x = x.astype(f32)
    x2 = jnp.transpose(jnp.reshape(x, (N, 2, HALF)), (1, 0, 2))
    xf = jnp.reshape(x2, (2 * N, HALF))
    zrows = jnp.zeros((RPT, HALF), f32)
    z16 = jnp.zeros((RPT, 16), f32)
    ones16 = jnp.ones((CH, 16), f32)
    pair0 = pair_index[0].astype(i32)
    pair1 = pair_index[1].astype(i32)
    eid = edge_id.astype(i32)

    outs = []
    for i in range(C):
        src = edges[i, 0].astype(i32)
        dst = edges[i, 1].astype(i32)
        srcg = jnp.concatenate([src, src + N])
        dstg = jnp.concatenate([dst, dst + N])
        srcs = src
        dsts = dst
        degt, cntt = _sc_degcnt(srcs, dsts, ones16, z16)
        h2 = x2
        hf = xf
        hsf = jnp.reshape(_tc_prep(x2, degt), (2 * N, HALF))
        ys, sts = [], []
        for j in range(L):
            uf, gf = _sc_agg(hsf, hf, srcg, dstg, srcs, dsts, zrows)
            tmp2, hs2, y, st = _tc_dense(
                h2, jnp.reshape(uf, (2, N, HALF)), jnp.reshape(gf, (2, N, HALF)),
                degt, cntt,
                W_amp[i, j], W_dgcn[i, j], W_diff[i, j], Wg1[i, j],
                jnp.reshape(Wg2[i, j], (1, D)), W_lin[i, j],
                jnp.reshape(b_amp[i, j], (1, D)), jnp.reshape(b_dgcn[i, j], (1, D)),
                jnp.reshape(b_diff[i, j], (1, D)), jnp.reshape(bg1[i, j], (1, D)),
                jnp.reshape(bg2[i, j], (1, 1)), jnp.reshape(b_lin[i, j], (1, H)))
            h2 = tmp2
            hf = jnp.reshape(tmp2, (2 * N, HALF))
            hsf = jnp.reshape(hs2, (2 * N, HALF))
            ys.append(y)
            sts.append(st)
        xa, xb = _sc_cls(pair0, pair1, eid, ys[0], ys[1], ys[2])
        o = _tc_cls(xa, xb, sts[0], sts[1], sts[2], gamma[i], beta[i],
                    W_fc1[i], jnp.reshape(b_fc1[i], (1, F2)),
                    W_fc2[i], jnp.reshape(b_fc2[i], (1, F3)),
                    W_fc3[i], jnp.reshape(b_fc3[i], (1, OUT)))
        outs.append(o)
    rcat = jnp.concatenate(outs, axis=1)
    return _tc_final(rcat, W_cls, jnp.reshape(b_cls, (1, C)))
